# baseline (speedup 1.0000x reference)
import functools

import jax
import jax.numpy as jnp
import numpy as np
from jax import lax
from jax.experimental import pallas as pl
from jax.experimental.pallas import tpu as pltpu
from jax.experimental.pallas import tpu_sc as plsc

F32 = jnp.float32

_N = 10000
_E = 320000
_NH = 2
_EH = _E // _NH
_NS = 128
_NP = 10240
_VW = 16
_SUB = 40
_NSUB = 5
_GROUP = _SUB * _NSUB
_NGTOT = _E // _GROUP
_NCORES = 2
_NTILES = 16
_NWORK = _NCORES * _NTILES
_EPW = _EH // _NWORK
_NGRP = _EPW // _GROUP
_ROWS_PT = _NP // _NTILES
_BE = 3200
_BN = 1280

_MESH = dict(core_axis_name="c", subcore_axis_name="s",
             num_cores=_NCORES, num_subcores=_NTILES)


def _prep_body(ns_ref, nv_ref, a_ref, b_ref, dm_ref, wht_ref, wvt_ref,
               ts_ref, td_ref, tu_ref):
    ns = ns_ref[...]
    nv = nv_ref[...]
    nh = jnp.dot(nv, wht_ref[...], preferred_element_type=F32)
    anorm = jnp.sqrt(jnp.sum(nh * nh, axis=-1, keepdims=True))
    u = jnp.dot(nh, wvt_ref[...], preferred_element_type=F32)
    ts_ref[...] = jnp.dot(ns, a_ref[...], preferred_element_type=F32) \
        + anorm * dm_ref[0:1, :]
    td_ref[...] = jnp.dot(ns, b_ref[...], preferred_element_type=F32) \
        + anorm * dm_ref[1:2, :]
    tu_ref[:, 0:3] = u
    tu_ref[:, 3:_VW] = jnp.zeros((ns.shape[0], _VW - 3), F32)


_prep_call = pl.pallas_call(
    _prep_body,
    grid=(_NP // _BN,),
    in_specs=[
        pl.BlockSpec((_BN, _NS), lambda i: (i, 0)),
        pl.BlockSpec((_BN, 3), lambda i: (i, 0)),
        pl.BlockSpec((_NS, _NS), lambda i: (0, 0)),
        pl.BlockSpec((_NS, _NS), lambda i: (0, 0)),
        pl.BlockSpec((3, _NS), lambda i: (0, 0)),
        pl.BlockSpec((3, 3), lambda i: (0, 0)),
        pl.BlockSpec((3, 3), lambda i: (0, 0)),
    ],
    out_specs=[pl.BlockSpec((_BN, _NS), lambda i: (i, 0)),
               pl.BlockSpec((_BN, _NS), lambda i: (i, 0)),
               pl.BlockSpec((_BN, _VW), lambda i: (i, 0))],
    out_shape=[jax.ShapeDtypeStruct((_NP, _NS), F32),
               jax.ShapeDtypeStruct((_NP, _NS), F32),
               jax.ShapeDtypeStruct((_NP, _VW), F32)],
)


@functools.cache
def _gather1_call(h):
    def body_fn(ts_hbm, td_hbm, si3_hbm, di3_hbm, gs_hbm, gd_hbm,
                sidx, didx, bs, bd, sem):
        c = lax.axis_index("c")
        s = lax.axis_index("s")
        wid = s * _NCORES + c

        def body(g, carry):
            base = wid * _EPW + g * _GROUP
            gid = h * (_EH // _GROUP) + wid * _NGRP + g
            pltpu.sync_copy(si3_hbm.at[gid], sidx)
            pltpu.sync_copy(di3_hbm.at[gid], didx)
            cps = []
            for j in range(_NSUB):
                cps.append(pltpu.async_copy(
                    ts_hbm.at[sidx.at[j]],
                    bs.at[pl.ds(j * _SUB, _SUB)], sem))
                cps.append(pltpu.async_copy(
                    td_hbm.at[didx.at[j]],
                    bd.at[pl.ds(j * _SUB, _SUB)], sem))
            for cp in cps:
                cp.wait()
            pltpu.sync_copy(bs, gs_hbm.at[pl.ds(base, _GROUP)])
            pltpu.sync_copy(bd, gd_hbm.at[pl.ds(base, _GROUP)])
            return carry

        lax.fori_loop(0, _NGRP, body, 0)

    return pl.kernel(
        body_fn,
        out_type=(jax.ShapeDtypeStruct((_EH, _NS), F32),
                  jax.ShapeDtypeStruct((_EH, _NS), F32)),
        mesh=plsc.VectorSubcoreMesh(**_MESH),
        scratch_types=[
            pltpu.VMEM((_NSUB, _SUB), jnp.int32),
            pltpu.VMEM((_NSUB, _SUB), jnp.int32),
            pltpu.VMEM((_GROUP, _NS), F32),
            pltpu.VMEM((_GROUP, _NS), F32),
            pltpu.SemaphoreType.DMA,
        ],
    )


@functools.cache
def _gather2_call(h):
    def body_fn(tu_hbm, si3_hbm, us_hbm, sidx, bu, sem):
        c = lax.axis_index("c")
        s = lax.axis_index("s")
        wid = s * _NCORES + c

        def body(g, carry):
            base = wid * _EPW + g * _GROUP
            gid = h * (_EH // _GROUP) + wid * _NGRP + g
            pltpu.sync_copy(si3_hbm.at[gid], sidx)
            cps = []
            for j in range(_NSUB):
                cps.append(pltpu.async_copy(
                    tu_hbm.at[sidx.at[j]],
                    bu.at[pl.ds(j * _SUB, _SUB)], sem))
            for cp in cps:
                cp.wait()
            pltpu.sync_copy(bu, us_hbm.at[pl.ds(base, _GROUP)])
            return carry

        lax.fori_loop(0, _NGRP, body, 0)

    return pl.kernel(
        body_fn,
        out_type=jax.ShapeDtypeStruct((_EH, _VW), F32),
        mesh=plsc.VectorSubcoreMesh(**_MESH),
        scratch_types=[
            pltpu.VMEM((_NSUB, _SUB), jnp.int32),
            pltpu.VMEM((_GROUP, _VW), F32),
            pltpu.SemaphoreType.DMA,
        ],
        compiler_params=pltpu.CompilerParams(use_tc_tiling_on_sc=False),
    )


def _edge_body(gs_ref, gd_ref, us_ref, es_ref, ev_ref, c16_ref, dm2_ref,
               bias_ref, wg8_ref, wgb8_ref, wht_ref, wc_ref,
               ms_ref, mv_ref):
    ev = ev_ref[...]
    vh = jnp.dot(ev, wht_ref[...], preferred_element_type=F32)
    cnorm = jnp.sqrt(jnp.sum(vh * vh, axis=-1, keepdims=True))
    evp = jnp.dot(ev, wc_ref[...], preferred_element_type=F32)
    q = jnp.dot(es_ref[...], c16_ref[...], preferred_element_type=F32)
    slin = (gs_ref[...] + gd_ref[...] + q
            + cnorm * dm2_ref[...] + bias_ref[...])
    so = jnp.maximum(slin, 0.0)
    gate = jax.nn.sigmoid(
        jnp.dot(so, wg8_ref[...], preferred_element_type=F32) + wgb8_ref[...])
    r0 = gate[:, 0:1] * us_ref[:, 0:3]
    r2 = gate[:, 2:3] * evp
    ms_ref[...] = so
    mv_ref[:, 0:3] = r0
    mv_ref[:, 3:4] = gate[:, 1:2]
    mv_ref[:, 4:7] = r2
    mv_ref[:, 7:_VW] = jnp.zeros((so.shape[0], _VW - 7), F32)


_edge_call = pl.pallas_call(
    _edge_body,
    grid=(_EH // _BE,),
    in_specs=[
        pl.BlockSpec((_BE, _NS), lambda i: (i, 0)),
        pl.BlockSpec((_BE, _NS), lambda i: (i, 0)),
        pl.BlockSpec((_BE, _VW), lambda i: (i, 0)),
        pl.BlockSpec((_BE, 16), lambda i: (i, 0)),
        pl.BlockSpec((_BE, 3), lambda i: (i, 0)),
        pl.BlockSpec((16, _NS), lambda i: (0, 0)),
        pl.BlockSpec((1, _NS), lambda i: (0, 0)),
        pl.BlockSpec((1, _NS), lambda i: (0, 0)),
        pl.BlockSpec((_NS, 8), lambda i: (0, 0)),
        pl.BlockSpec((1, 8), lambda i: (0, 0)),
        pl.BlockSpec((3, 3), lambda i: (0, 0)),
        pl.BlockSpec((3, 3), lambda i: (0, 0)),
    ],
    out_specs=[pl.BlockSpec((_BE, _NS), lambda i: (i, 0)),
               pl.BlockSpec((_BE, _VW), lambda i: (i, 0))],
    out_shape=[jax.ShapeDtypeStruct((_EH, _NS), F32),
               jax.ShapeDtypeStruct((_EH, _VW), F32)],
)


@functools.cache
def _scatter1_call(h):
    def body_fn(m_hbm, di3_hbm, z_hbm, out_hbm, didx, buf, acc, sem):
        c = lax.axis_index("c")
        s = lax.axis_index("s")
        pltpu.sync_copy(z_hbm.at[pl.ds(s * _ROWS_PT, _ROWS_PT)],
                        acc.at[pl.ds(s * _ROWS_PT, _ROWS_PT)])
        plsc.subcore_barrier()
        base0 = c * (_EH // _NCORES) + s * _EPW

        def body(g, carry):
            base = base0 + g * _GROUP
            gid = (h * _EH + base0) // _GROUP + g
            pltpu.sync_copy(di3_hbm.at[gid], didx)
            pltpu.sync_copy(m_hbm.at[pl.ds(base, _GROUP)], buf)
            cps = []
            for j in range(_NSUB):
                cps.append(pltpu.async_copy(
                    buf.at[pl.ds(j * _SUB, _SUB)], acc.at[didx.at[j]], sem,
                    add=True))
            for cp in cps:
                cp.wait()
            return carry

        lax.fori_loop(0, _NGRP, body, 0)
        plsc.subcore_barrier()
        pltpu.sync_copy(acc.at[pl.ds(s * _ROWS_PT, _ROWS_PT)],
                        out_hbm.at[c, pl.ds(s * _ROWS_PT, _ROWS_PT)])

    return pl.kernel(
        body_fn,
        out_type=jax.ShapeDtypeStruct((_NCORES, _NP, _NS), F32),
        mesh=plsc.VectorSubcoreMesh(**_MESH),
        scratch_types=[
            pltpu.VMEM((_NSUB, _SUB), jnp.int32),
            pltpu.VMEM((_GROUP, _NS), F32),
            pltpu.VMEM_SHARED((_NP, _NS), F32),
            pltpu.SemaphoreType.DMA,
        ],
    )


@functools.cache
def _scatter2_call(h):
    def body_fn(m_hbm, di3_hbm, z_hbm, out_hbm, didx, buf, acc, sem):
        c = lax.axis_index("c")
        s = lax.axis_index("s")
        pltpu.sync_copy(z_hbm.at[pl.ds(s * _ROWS_PT, _ROWS_PT)],
                        acc.at[pl.ds(s * _ROWS_PT, _ROWS_PT)])
        plsc.subcore_barrier()
        base0 = c * (_EH // _NCORES) + s * _EPW

        def body(g, carry):
            base = base0 + g * _GROUP
            gid = (h * _EH + base0) // _GROUP + g
            pltpu.sync_copy(di3_hbm.at[gid], didx)
            pltpu.sync_copy(m_hbm.at[pl.ds(base, _GROUP)], buf)
            cps = []
            for j in range(_NSUB):
                cps.append(pltpu.async_copy(
                    buf.at[pl.ds(j * _SUB, _SUB)], acc.at[didx.at[j]], sem,
                    add=True))
            for cp in cps:
                cp.wait()
            return carry

        lax.fori_loop(0, _NGRP, body, 0)
        plsc.subcore_barrier()
        pltpu.sync_copy(acc.at[pl.ds(s * _ROWS_PT, _ROWS_PT)],
                        out_hbm.at[c, pl.ds(s * _ROWS_PT, _ROWS_PT)])

    return pl.kernel(
        body_fn,
        out_type=jax.ShapeDtypeStruct((_NCORES, _NP, _VW), F32),
        mesh=plsc.VectorSubcoreMesh(**_MESH),
        scratch_types=[
            pltpu.VMEM((_NSUB, _SUB), jnp.int32),
            pltpu.VMEM((_GROUP, _VW), F32),
            pltpu.VMEM_SHARED((_NP, _VW), F32),
            pltpu.SemaphoreType.DMA,
        ],
        compiler_params=pltpu.CompilerParams(use_tc_tiling_on_sc=False),
    )


def _node_gvp(s, v9, a_ref, d_ref, b_ref, wg8_ref, wgb8_ref, bdh_ref,
              bdc_ref, sel_ref, selt_ref):
    vh9 = jnp.dot(v9, bdh_ref[...], preferred_element_type=F32)
    vn = jnp.sqrt(jnp.dot(vh9 * vh9, sel_ref[...],
                          preferred_element_type=F32))
    slin = (jnp.dot(s, a_ref[...], preferred_element_type=F32)
            + jnp.dot(vn, d_ref[...], preferred_element_type=F32)
            + b_ref[...])
    so = jnp.maximum(slin, 0.0)
    gate = jax.nn.sigmoid(
        jnp.dot(so, wg8_ref[...], preferred_element_type=F32)
        + wgb8_ref[...])[:, 0:3]
    gate9 = jnp.dot(gate, selt_ref[...], preferred_element_type=F32)
    vout = jnp.dot(v9, bdc_ref[...], preferred_element_type=F32) * gate9
    return so, vout


def _layernorm(x, w, b):
    mu = jnp.mean(x, axis=-1, keepdims=True)
    var = jnp.mean((x - mu) ** 2, axis=-1, keepdims=True)
    return (x - mu) / jnp.sqrt(var + 1e-5) * w + b


def _node_body(pa0_ref, pa1_ref, pb0_ref, pb1_ref,
               qa0_ref, qa1_ref, qb0_ref, qb1_ref, ns_ref, nv_ref,
               wht_ref, wvt_ref,
               ln1w_ref, ln1b_ref, ln2w_ref, ln2b_ref,
               a0_ref, d0_ref, b0_ref, wg0_ref, wgb0_ref, bdh0_ref, bdc0_ref,
               a1_ref, d1_ref, b1_ref, wg1_ref, wgb1_ref, bdh1_ref, bdc1_ref,
               sel_ref, selt_ref, os_ref, ov_ref):
    agg_s = (pa0_ref[...] + pa1_ref[...] + pb0_ref[...] + pb1_ref[...]
             + ns_ref[...])
    pv = qa0_ref[...] + qa1_ref[...] + qb0_ref[...] + qb1_ref[...]
    nh = jnp.dot(nv_ref[...], wht_ref[...], preferred_element_type=F32)
    u = jnp.dot(nh, wvt_ref[...], preferred_element_type=F32)
    v9 = jnp.concatenate(
        [pv[:, 0:3], u * pv[:, 3:4], pv[:, 4:7]], axis=1)
    s1 = _layernorm(agg_s, ln1w_ref[...], ln1b_ref[...])
    rms = jnp.sqrt(jnp.mean(v9 * v9, axis=-1, keepdims=True) + 1e-8)
    v1 = v9 / rms
    s2, v2 = _node_gvp(s1, v1, a0_ref, d0_ref, b0_ref, wg0_ref, wgb0_ref,
                       bdh0_ref, bdc0_ref, sel_ref, selt_ref)
    s3, v3 = _node_gvp(s2, v2, a1_ref, d1_ref, b1_ref, wg1_ref, wgb1_ref,
                       bdh1_ref, bdc1_ref, sel_ref, selt_ref)
    o_s = s1 + s3
    o_v = v1 + v3
    os_ref[...] = _layernorm(o_s, ln2w_ref[...], ln2b_ref[...])
    rms2 = jnp.sqrt(jnp.mean(o_v * o_v, axis=-1, keepdims=True) + 1e-8)
    ov_ref[...] = o_v / rms2


def _full(shape):
    return pl.BlockSpec(shape, lambda i: tuple(0 for _ in shape))


_node_call = pl.pallas_call(
    _node_body,
    grid=(_NP // _BN,),
    in_specs=[
        pl.BlockSpec((_BN, _NS), lambda i: (i, 0)),
        pl.BlockSpec((_BN, _NS), lambda i: (i, 0)),
        pl.BlockSpec((_BN, _NS), lambda i: (i, 0)),
        pl.BlockSpec((_BN, _NS), lambda i: (i, 0)),
        pl.BlockSpec((_BN, _VW), lambda i: (i, 0)),
        pl.BlockSpec((_BN, _VW), lambda i: (i, 0)),
        pl.BlockSpec((_BN, _VW), lambda i: (i, 0)),
        pl.BlockSpec((_BN, _VW), lambda i: (i, 0)),
        pl.BlockSpec((_BN, _NS), lambda i: (i, 0)),
        pl.BlockSpec((_BN, 3), lambda i: (i, 0)),
        _full((3, 3)), _full((3, 3)),
        _full((1, _NS)), _full((1, _NS)), _full((1, _NS)), _full((1, _NS)),
        _full((_NS, _NS)), _full((3, _NS)), _full((1, _NS)),
        _full((_NS, 8)), _full((1, 8)), _full((9, 9)), _full((9, 9)),
        _full((_NS, _NS)), _full((3, _NS)), _full((1, _NS)),
        _full((_NS, 8)), _full((1, 8)), _full((9, 9)), _full((9, 9)),
        _full((9, 3)), _full((3, 9)),
    ],
    out_specs=[pl.BlockSpec((_BN, _NS), lambda i: (i, 0)),
               pl.BlockSpec((_BN, 9), lambda i: (i, 0))],
    out_shape=[jax.ShapeDtypeStruct((_NP, _NS), F32),
               jax.ShapeDtypeStruct((_NP, 9), F32)],
)

_SEL = np.zeros((9, 3), np.float32)
for _i in range(3):
    for _k in range(3):
        _SEL[3 * _i + _k, _i] = 1.0


def _blockdiag3(w):
    z = jnp.zeros((9, 9), F32)
    for i in range(3):
        z = z.at[3 * i:3 * i + 3, 3 * i:3 * i + 3].set(w)
    return z


def _pad8(w3):
    return jnp.zeros((w3.shape[1], 8), F32).at[:, 0:3].set(w3.T)


def kernel(node_s, node_v, edge_s, edge_v, msg_Wh, msg_WV, msg_Ws_w,
           msg_Ws_b, msg_Wg_w, msg_Wg_b, ff0_Wh, ff0_WV, ff0_Ws_w, ff0_Ws_b,
           ff0_Wg_w, ff0_Wg_b, ff1_Wh, ff1_WV, ff1_Ws_w, ff1_Ws_b, ff1_Wg_w,
           ff1_Wg_b, ln1_w, ln1_b, ln2_w, ln2_b, edge_index):
    ns_p = jnp.zeros((_NP, _NS), F32).at[:_N].set(node_s)
    nv_p = jnp.zeros((_NP, 3), F32).at[:_N].set(node_v.reshape(_N, 3))
    wst = msg_Ws_w.T
    a_w, b_w, c16, dm = wst[0:128], wst[128:256], wst[256:272], wst[272:275]
    wht = msg_Wh.T
    wc = msg_Wh.T @ msg_WV.T
    ts, td, tu = _prep_call(ns_p, nv_p, a_w, b_w, dm, wht, msg_WV.T)
    src3 = edge_index[0].reshape(_NGTOT, _NSUB, _SUB)
    dst3 = edge_index[1].reshape(_NGTOT, _NSUB, _SUB)
    ev3 = edge_v.reshape(_E, 3)
    wgb8 = jnp.zeros((1, 8), F32).at[0, 0:3].set(msg_Wg_b)
    wg8 = _pad8(msg_Wg_w)
    zeros_s = jnp.zeros((_NP, _NS), F32)
    zeros_v = jnp.zeros((_NP, _VW), F32)

    gath, usv, msv, mvv, ps, pv = [], [], [], [], [], []
    for hh in range(_NH):
        gath.append(_gather1_call(hh)(ts, td, src3, dst3))
        usv.append(_gather2_call(hh)(tu, src3))
    for hh in range(_NH):
        gs, gd = gath[hh]
        sl = slice(hh * _EH, (hh + 1) * _EH)
        m_s, m_v = _edge_call(gs, gd, usv[hh], edge_s[sl], ev3[sl], c16,
                              dm[2:3], msg_Ws_b[None], wg8, wgb8, wht, wc)
        msv.append(m_s)
        mvv.append(m_v)
    for hh in range(_NH):
        ps.append(_scatter1_call(hh)(msv[hh], dst3, zeros_s))
        pv.append(_scatter2_call(hh)(mvv[hh], dst3, zeros_v))

    def ffw(ws_w, ws_b, wg_w, wg_b, wh, wv):
        t = ws_w.T
        wgb = jnp.zeros((1, 8), F32).at[0, 0:3].set(wg_b)
        return (t[0:128], t[128:131], ws_b[None], _pad8(wg_w), wgb,
                _blockdiag3(wh.T), _blockdiag3(wh.T @ wv.T))

    sel = jnp.asarray(_SEL)
    out_s, out_v9 = _node_call(
        ps[0][0], ps[0][1], ps[1][0], ps[1][1],
        pv[0][0], pv[0][1], pv[1][0], pv[1][1], ns_p, nv_p,
        wht, msg_WV.T,
        ln1_w[None], ln1_b[None], ln2_w[None], ln2_b[None],
        *ffw(ff0_Ws_w, ff0_Ws_b, ff0_Wg_w, ff0_Wg_b, ff0_Wh, ff0_WV),
        *ffw(ff1_Ws_w, ff1_Ws_b, ff1_Wg_w, ff1_Wg_b, ff1_Wh, ff1_WV),
        sel, sel.T)
    return out_s[:_N], out_v9[:_N].reshape(_N, 3, 3)


# speedup vs baseline: 1.1016x; 1.1016x over previous
import functools

import jax
import jax.numpy as jnp
import numpy as np
from jax import lax
from jax.experimental import pallas as pl
from jax.experimental.pallas import tpu as pltpu
from jax.experimental.pallas import tpu_sc as plsc

F32 = jnp.float32

_N = 10000
_E = 320000
_NH = 1
_EH = _E // _NH
_NS = 128
_NP = 10240
_VW = 16
_SUB = 40
_NSUB = 5
_GROUP = _SUB * _NSUB
_NGTOT = _E // _GROUP
_NCORES = 2
_NTILES = 16
_NWORK = _NCORES * _NTILES
_EPW = _EH // _NWORK
_NGRP = _EPW // _GROUP
_ROWS_PT = _NP // _NTILES
_BE = 3200
_BN = 1280

_MESH = dict(core_axis_name="c", subcore_axis_name="s",
             num_cores=_NCORES, num_subcores=_NTILES)


def _prep_body(ns_ref, nv_ref, a_ref, b_ref, dm_ref, wht_ref, wvt_ref,
               bias_ref, ts_ref, td_ref, tu_ref):
    ns = ns_ref[...]
    nv = nv_ref[...]
    nh = jnp.dot(nv, wht_ref[...], preferred_element_type=F32)
    anorm = jnp.sqrt(jnp.sum(nh * nh, axis=-1, keepdims=True))
    u = jnp.dot(nh, wvt_ref[...], preferred_element_type=F32)
    ts_ref[...] = jnp.dot(ns, a_ref[...], preferred_element_type=F32) \
        + anorm * dm_ref[0:1, :] + bias_ref[...]
    td_ref[...] = jnp.dot(ns, b_ref[...], preferred_element_type=F32) \
        + anorm * dm_ref[1:2, :]
    tu_ref[:, 0:3] = u
    tu_ref[:, 3:_VW] = jnp.zeros((ns.shape[0], _VW - 3), F32)


_prep_call = pl.pallas_call(
    _prep_body,
    grid=(_NP // _BN,),
    in_specs=[
        pl.BlockSpec((_BN, _NS), lambda i: (i, 0)),
        pl.BlockSpec((_BN, 3), lambda i: (i, 0)),
        pl.BlockSpec((_NS, _NS), lambda i: (0, 0)),
        pl.BlockSpec((_NS, _NS), lambda i: (0, 0)),
        pl.BlockSpec((3, _NS), lambda i: (0, 0)),
        pl.BlockSpec((3, 3), lambda i: (0, 0)),
        pl.BlockSpec((3, 3), lambda i: (0, 0)),
        pl.BlockSpec((1, _NS), lambda i: (0, 0)),
    ],
    out_specs=[pl.BlockSpec((_BN, _NS), lambda i: (i, 0)),
               pl.BlockSpec((_BN, _NS), lambda i: (i, 0)),
               pl.BlockSpec((_BN, _VW), lambda i: (i, 0))],
    out_shape=[jax.ShapeDtypeStruct((_NP, _NS), F32),
               jax.ShapeDtypeStruct((_NP, _NS), F32),
               jax.ShapeDtypeStruct((_NP, _VW), F32)],
)


@functools.cache
def _gather1_call(h):
    def body_fn(ts_hbm, td_hbm, si3_hbm, di3_hbm, gs_hbm, gd_hbm,
                sidx, didx, bs, bd, sem):
        c = lax.axis_index("c")
        s = lax.axis_index("s")
        wid = s * _NCORES + c

        def body(g, carry):
            base = wid * _EPW + g * _GROUP
            gid = h * (_EH // _GROUP) + wid * _NGRP + g
            pltpu.sync_copy(si3_hbm.at[gid], sidx)
            pltpu.sync_copy(di3_hbm.at[gid], didx)
            cps = []
            for j in range(_NSUB):
                cps.append(pltpu.async_copy(
                    ts_hbm.at[sidx.at[j]],
                    bs.at[pl.ds(j * _SUB, _SUB)], sem))
                cps.append(pltpu.async_copy(
                    td_hbm.at[didx.at[j]],
                    bd.at[pl.ds(j * _SUB, _SUB)], sem))
            for cp in cps:
                cp.wait()
            pltpu.sync_copy(bs, gs_hbm.at[pl.ds(base, _GROUP)])
            pltpu.sync_copy(bd, gd_hbm.at[pl.ds(base, _GROUP)])
            return carry

        lax.fori_loop(0, _NGRP, body, 0)

    return pl.kernel(
        body_fn,
        out_type=(jax.ShapeDtypeStruct((_EH, _NS), F32),
                  jax.ShapeDtypeStruct((_EH, _NS), F32)),
        mesh=plsc.VectorSubcoreMesh(**_MESH),
        scratch_types=[
            pltpu.VMEM((_NSUB, _SUB), jnp.int32),
            pltpu.VMEM((_NSUB, _SUB), jnp.int32),
            pltpu.VMEM((_GROUP, _NS), F32),
            pltpu.VMEM((_GROUP, _NS), F32),
            pltpu.SemaphoreType.DMA,
        ],
    )


@functools.cache
def _gather2_call(h):
    def body_fn(tu_hbm, si3_hbm, us_hbm, sidx, bu, sem):
        c = lax.axis_index("c")
        s = lax.axis_index("s")
        wid = s * _NCORES + c

        def body(g, carry):
            base = wid * _EPW + g * _GROUP
            gid = h * (_EH // _GROUP) + wid * _NGRP + g
            pltpu.sync_copy(si3_hbm.at[gid], sidx)
            cps = []
            for j in range(_NSUB):
                cps.append(pltpu.async_copy(
                    tu_hbm.at[sidx.at[j]],
                    bu.at[pl.ds(j * _SUB, _SUB)], sem))
            for cp in cps:
                cp.wait()
            pltpu.sync_copy(bu, us_hbm.at[pl.ds(base, _GROUP)])
            return carry

        lax.fori_loop(0, _NGRP, body, 0)

    return pl.kernel(
        body_fn,
        out_type=jax.ShapeDtypeStruct((_EH, _VW), F32),
        mesh=plsc.VectorSubcoreMesh(**_MESH),
        scratch_types=[
            pltpu.VMEM((_NSUB, _SUB), jnp.int32),
            pltpu.VMEM((_GROUP, _VW), F32),
            pltpu.SemaphoreType.DMA,
        ],
        compiler_params=pltpu.CompilerParams(use_tc_tiling_on_sc=False),
    )


def _edge_body(gs_ref, gd_ref, us_ref, es_ref, ev_ref, c16_ref, dm2_ref,
               wg8_ref, wgb8_ref, wht_ref, wcm_ref, e1_ref, c3_ref,
               ms_ref, mv_ref):
    ev = ev_ref[...]
    vh = jnp.dot(ev, wht_ref[...], preferred_element_type=F32)
    cnorm = jnp.sqrt(jnp.sum(vh * vh, axis=-1, keepdims=True))
    q = jnp.dot(es_ref[...], c16_ref[...], preferred_element_type=F32)
    slin = gs_ref[...] + gd_ref[...] + q + cnorm * dm2_ref[...]
    so = jnp.maximum(slin, 0.0)
    gate = jax.nn.sigmoid(
        jnp.dot(so, wg8_ref[...], preferred_element_type=F32) + wgb8_ref[...])
    mult = (us_ref[...]
            + jnp.dot(ev, wcm_ref[...], preferred_element_type=F32)
            + c3_ref[...])
    gate16 = jnp.dot(gate, e1_ref[...], preferred_element_type=F32)
    ms_ref[...] = so
    mv_ref[...] = gate16 * mult


_edge_call = pl.pallas_call(
    _edge_body,
    grid=(_EH // _BE,),
    in_specs=[
        pl.BlockSpec((_BE, _NS), lambda i: (i, 0)),
        pl.BlockSpec((_BE, _NS), lambda i: (i, 0)),
        pl.BlockSpec((_BE, _VW), lambda i: (i, 0)),
        pl.BlockSpec((_BE, 16), lambda i: (i, 0)),
        pl.BlockSpec((_BE, 3), lambda i: (i, 0)),
        pl.BlockSpec((16, _NS), lambda i: (0, 0)),
        pl.BlockSpec((1, _NS), lambda i: (0, 0)),
        pl.BlockSpec((_NS, 8), lambda i: (0, 0)),
        pl.BlockSpec((1, 8), lambda i: (0, 0)),
        pl.BlockSpec((3, 3), lambda i: (0, 0)),
        pl.BlockSpec((3, _VW), lambda i: (0, 0)),
        pl.BlockSpec((8, _VW), lambda i: (0, 0)),
        pl.BlockSpec((1, _VW), lambda i: (0, 0)),
    ],
    out_specs=[pl.BlockSpec((_BE, _NS), lambda i: (i, 0)),
               pl.BlockSpec((_BE, _VW), lambda i: (i, 0))],
    out_shape=[jax.ShapeDtypeStruct((_EH, _NS), F32),
               jax.ShapeDtypeStruct((_EH, _VW), F32)],
)


@functools.cache
def _scatter1_call(h):
    def body_fn(m_hbm, di3_hbm, z_hbm, out_hbm, didx, buf, acc, sem):
        c = lax.axis_index("c")
        s = lax.axis_index("s")
        pltpu.sync_copy(z_hbm.at[pl.ds(s * _ROWS_PT, _ROWS_PT)],
                        acc.at[pl.ds(s * _ROWS_PT, _ROWS_PT)])
        plsc.subcore_barrier()
        base0 = c * (_EH // _NCORES) + s * _EPW

        def body(g, carry):
            base = base0 + g * _GROUP
            gid = (h * _EH + base0) // _GROUP + g
            pltpu.sync_copy(di3_hbm.at[gid], didx)
            pltpu.sync_copy(m_hbm.at[pl.ds(base, _GROUP)], buf)
            cps = []
            for j in range(_NSUB):
                cps.append(pltpu.async_copy(
                    buf.at[pl.ds(j * _SUB, _SUB)], acc.at[didx.at[j]], sem,
                    add=True))
            for cp in cps:
                cp.wait()
            return carry

        lax.fori_loop(0, _NGRP, body, 0)
        plsc.subcore_barrier()
        pltpu.sync_copy(acc.at[pl.ds(s * _ROWS_PT, _ROWS_PT)],
                        out_hbm.at[c, pl.ds(s * _ROWS_PT, _ROWS_PT)])

    return pl.kernel(
        body_fn,
        out_type=jax.ShapeDtypeStruct((_NCORES, _NP, _NS), F32),
        mesh=plsc.VectorSubcoreMesh(**_MESH),
        scratch_types=[
            pltpu.VMEM((_NSUB, _SUB), jnp.int32),
            pltpu.VMEM((_GROUP, _NS), F32),
            pltpu.VMEM_SHARED((_NP, _NS), F32),
            pltpu.SemaphoreType.DMA,
        ],
    )


@functools.cache
def _scatter2_call(h):
    def body_fn(m_hbm, di3_hbm, z_hbm, out_hbm, didx, buf, acc, sem):
        c = lax.axis_index("c")
        s = lax.axis_index("s")
        pltpu.sync_copy(z_hbm.at[pl.ds(s * _ROWS_PT, _ROWS_PT)],
                        acc.at[pl.ds(s * _ROWS_PT, _ROWS_PT)])
        plsc.subcore_barrier()
        base0 = c * (_EH // _NCORES) + s * _EPW

        def body(g, carry):
            base = base0 + g * _GROUP
            gid = (h * _EH + base0) // _GROUP + g
            pltpu.sync_copy(di3_hbm.at[gid], didx)
            pltpu.sync_copy(m_hbm.at[pl.ds(base, _GROUP)], buf)
            cps = []
            for j in range(_NSUB):
                cps.append(pltpu.async_copy(
                    buf.at[pl.ds(j * _SUB, _SUB)], acc.at[didx.at[j]], sem,
                    add=True))
            for cp in cps:
                cp.wait()
            return carry

        lax.fori_loop(0, _NGRP, body, 0)
        plsc.subcore_barrier()
        pltpu.sync_copy(acc.at[pl.ds(s * _ROWS_PT, _ROWS_PT)],
                        out_hbm.at[c, pl.ds(s * _ROWS_PT, _ROWS_PT)])

    return pl.kernel(
        body_fn,
        out_type=jax.ShapeDtypeStruct((_NCORES, _NP, _VW), F32),
        mesh=plsc.VectorSubcoreMesh(**_MESH),
        scratch_types=[
            pltpu.VMEM((_NSUB, _SUB), jnp.int32),
            pltpu.VMEM((_GROUP, _VW), F32),
            pltpu.VMEM_SHARED((_NP, _VW), F32),
            pltpu.SemaphoreType.DMA,
        ],
        compiler_params=pltpu.CompilerParams(use_tc_tiling_on_sc=False),
    )


def _node_gvp(s, v9, a_ref, d_ref, b_ref, wg8_ref, wgb8_ref, bdh_ref,
              bdc_ref, sel_ref, selt_ref):
    vh9 = jnp.dot(v9, bdh_ref[...], preferred_element_type=F32)
    vn = jnp.sqrt(jnp.dot(vh9 * vh9, sel_ref[...],
                          preferred_element_type=F32))
    slin = (jnp.dot(s, a_ref[...], preferred_element_type=F32)
            + jnp.dot(vn, d_ref[...], preferred_element_type=F32)
            + b_ref[...])
    so = jnp.maximum(slin, 0.0)
    gate = jax.nn.sigmoid(
        jnp.dot(so, wg8_ref[...], preferred_element_type=F32)
        + wgb8_ref[...])[:, 0:3]
    gate9 = jnp.dot(gate, selt_ref[...], preferred_element_type=F32)
    vout = jnp.dot(v9, bdc_ref[...], preferred_element_type=F32) * gate9
    return so, vout


def _layernorm(x, w, b):
    mu = jnp.mean(x, axis=-1, keepdims=True)
    var = jnp.mean((x - mu) ** 2, axis=-1, keepdims=True)
    return (x - mu) / jnp.sqrt(var + 1e-5) * w + b


def _node_body(pa0_ref, pa1_ref,
               qa0_ref, qa1_ref, ns_ref, nv_ref,
               wht_ref, wvt_ref,
               ln1w_ref, ln1b_ref, ln2w_ref, ln2b_ref,
               a0_ref, d0_ref, b0_ref, wg0_ref, wgb0_ref, bdh0_ref, bdc0_ref,
               a1_ref, d1_ref, b1_ref, wg1_ref, wgb1_ref, bdh1_ref, bdc1_ref,
               sel_ref, selt_ref, os_ref, ov_ref):
    agg_s = pa0_ref[...] + pa1_ref[...] + ns_ref[...]
    pv = qa0_ref[...] + qa1_ref[...]
    nh = jnp.dot(nv_ref[...], wht_ref[...], preferred_element_type=F32)
    u = jnp.dot(nh, wvt_ref[...], preferred_element_type=F32)
    v9 = jnp.concatenate(
        [pv[:, 0:3], u * pv[:, 3:4], pv[:, 4:7]], axis=1)
    s1 = _layernorm(agg_s, ln1w_ref[...], ln1b_ref[...])
    rms = jnp.sqrt(jnp.mean(v9 * v9, axis=-1, keepdims=True) + 1e-8)
    v1 = v9 / rms
    s2, v2 = _node_gvp(s1, v1, a0_ref, d0_ref, b0_ref, wg0_ref, wgb0_ref,
                       bdh0_ref, bdc0_ref, sel_ref, selt_ref)
    s3, v3 = _node_gvp(s2, v2, a1_ref, d1_ref, b1_ref, wg1_ref, wgb1_ref,
                       bdh1_ref, bdc1_ref, sel_ref, selt_ref)
    o_s = s1 + s3
    o_v = v1 + v3
    os_ref[...] = _layernorm(o_s, ln2w_ref[...], ln2b_ref[...])
    rms2 = jnp.sqrt(jnp.mean(o_v * o_v, axis=-1, keepdims=True) + 1e-8)
    ov_ref[...] = o_v / rms2


def _full(shape):
    return pl.BlockSpec(shape, lambda i: tuple(0 for _ in shape))


_node_call = pl.pallas_call(
    _node_body,
    grid=(_NP // _BN,),
    in_specs=[
        pl.BlockSpec((_BN, _NS), lambda i: (i, 0)),
        pl.BlockSpec((_BN, _NS), lambda i: (i, 0)),
        pl.BlockSpec((_BN, _VW), lambda i: (i, 0)),
        pl.BlockSpec((_BN, _VW), lambda i: (i, 0)),
        pl.BlockSpec((_BN, _NS), lambda i: (i, 0)),
        pl.BlockSpec((_BN, 3), lambda i: (i, 0)),
        _full((3, 3)), _full((3, 3)),
        _full((1, _NS)), _full((1, _NS)), _full((1, _NS)), _full((1, _NS)),
        _full((_NS, _NS)), _full((3, _NS)), _full((1, _NS)),
        _full((_NS, 8)), _full((1, 8)), _full((9, 9)), _full((9, 9)),
        _full((_NS, _NS)), _full((3, _NS)), _full((1, _NS)),
        _full((_NS, 8)), _full((1, 8)), _full((9, 9)), _full((9, 9)),
        _full((9, 3)), _full((3, 9)),
    ],
    out_specs=[pl.BlockSpec((_BN, _NS), lambda i: (i, 0)),
               pl.BlockSpec((_BN, 9), lambda i: (i, 0))],
    out_shape=[jax.ShapeDtypeStruct((_NP, _NS), F32),
               jax.ShapeDtypeStruct((_NP, 9), F32)],
)

_SEL = np.zeros((9, 3), np.float32)
for _i in range(3):
    for _k in range(3):
        _SEL[3 * _i + _k, _i] = 1.0


def _blockdiag3(w):
    z = jnp.zeros((9, 9), F32)
    for i in range(3):
        z = z.at[3 * i:3 * i + 3, 3 * i:3 * i + 3].set(w)
    return z


def _pad8(w3):
    return jnp.zeros((w3.shape[1], 8), F32).at[:, 0:3].set(w3.T)


def kernel(node_s, node_v, edge_s, edge_v, msg_Wh, msg_WV, msg_Ws_w,
           msg_Ws_b, msg_Wg_w, msg_Wg_b, ff0_Wh, ff0_WV, ff0_Ws_w, ff0_Ws_b,
           ff0_Wg_w, ff0_Wg_b, ff1_Wh, ff1_WV, ff1_Ws_w, ff1_Ws_b, ff1_Wg_w,
           ff1_Wg_b, ln1_w, ln1_b, ln2_w, ln2_b, edge_index):
    ns_p = jnp.zeros((_NP, _NS), F32).at[:_N].set(node_s)
    nv_p = jnp.zeros((_NP, 3), F32).at[:_N].set(node_v.reshape(_N, 3))
    wst = msg_Ws_w.T
    a_w, b_w, c16, dm = wst[0:128], wst[128:256], wst[256:272], wst[272:275]
    wht = msg_Wh.T
    wc = msg_Wh.T @ msg_WV.T
    ts, td, tu = _prep_call(ns_p, nv_p, a_w, b_w, dm, wht, msg_WV.T,
                            msg_Ws_b[None])
    src3 = edge_index[0].reshape(_NGTOT, _NSUB, _SUB)
    dst3 = edge_index[1].reshape(_NGTOT, _NSUB, _SUB)
    ev3 = edge_v.reshape(_E, 3)
    wgb8 = jnp.zeros((1, 8), F32).at[0, 0:3].set(msg_Wg_b)
    wg8 = _pad8(msg_Wg_w)
    zeros_s = jnp.zeros((_NP, _NS), F32)
    zeros_v = jnp.zeros((_NP, _VW), F32)
    wcm = jnp.zeros((3, _VW), F32).at[:, 4:7].set(wc)
    c3 = jnp.zeros((1, _VW), F32).at[0, 3].set(1.0)
    e1 = jnp.zeros((8, _VW), F32)
    e1 = e1.at[0, 0:3].set(1.0).at[1, 3].set(1.0).at[2, 4:7].set(1.0)

    gath, usv, msv, mvv, ps, pv = [], [], [], [], [], []
    for hh in range(_NH):
        gath.append(_gather1_call(hh)(ts, td, src3, dst3))
        usv.append(_gather2_call(hh)(tu, src3))
    for hh in range(_NH):
        gs, gd = gath[hh]
        sl = slice(hh * _EH, (hh + 1) * _EH)
        m_s, m_v = _edge_call(gs, gd, usv[hh], edge_s[sl], ev3[sl], c16,
                              dm[2:3], wg8, wgb8, wht, wcm, e1, c3)
        msv.append(m_s)
        mvv.append(m_v)
    for hh in range(_NH):
        ps.append(_scatter1_call(hh)(msv[hh], dst3, zeros_s))
        pv.append(_scatter2_call(hh)(mvv[hh], dst3, zeros_v))

    def ffw(ws_w, ws_b, wg_w, wg_b, wh, wv):
        t = ws_w.T
        wgb = jnp.zeros((1, 8), F32).at[0, 0:3].set(wg_b)
        return (t[0:128], t[128:131], ws_b[None], _pad8(wg_w), wgb,
                _blockdiag3(wh.T), _blockdiag3(wh.T @ wv.T))

    sel = jnp.asarray(_SEL)
    out_s, out_v9 = _node_call(
        ps[0][0], ps[0][1],
        pv[0][0], pv[0][1], ns_p, nv_p,
        wht, msg_WV.T,
        ln1_w[None], ln1_b[None], ln2_w[None], ln2_b[None],
        *ffw(ff0_Ws_w, ff0_Ws_b, ff0_Wg_w, ff0_Wg_b, ff0_Wh, ff0_WV),
        *ffw(ff1_Ws_w, ff1_Ws_b, ff1_Wg_w, ff1_Wg_b, ff1_Wh, ff1_WV),
        sel, sel.T)
    return out_s[:_N], out_v9[:_N].reshape(_N, 3, 3)


# speedup vs baseline: 1.1568x; 1.0501x over previous
import functools

import jax
import jax.numpy as jnp
import numpy as np
from jax import lax
from jax.experimental import pallas as pl
from jax.experimental.pallas import tpu as pltpu
from jax.experimental.pallas import tpu_sc as plsc

F32 = jnp.float32

_N = 10000
_E = 320000
_NH = 1
_EH = _E // _NH
_NS = 128
_NP = 10240
_VW = 16
_SUB = 80
_NSUB = 5
_GROUP = _SUB * _NSUB
_NGTOT = _E // _GROUP
_S1SUB = 40
_S1NSUB = 5
_S1GROUP = _S1SUB * _S1NSUB
_NCORES = 2
_NTILES = 16
_NWORK = _NCORES * _NTILES
_EPW = _EH // _NWORK
_NGRP = _EPW // _GROUP
_ROWS_PT = _NP // _NTILES
_BE = 3200
_BN = 1280

_MESH = dict(core_axis_name="c", subcore_axis_name="s",
             num_cores=_NCORES, num_subcores=_NTILES)


def _prep_body(ns_ref, nv_ref, a_ref, b_ref, dm_ref, wht_ref, wvt_ref,
               bias_ref, ts_ref, td_ref, tu_ref):
    ns = ns_ref[...]
    nv = nv_ref[...]
    nh = jnp.dot(nv, wht_ref[...], preferred_element_type=F32)
    anorm = jnp.sqrt(jnp.sum(nh * nh, axis=-1, keepdims=True))
    u = jnp.dot(nh, wvt_ref[...], preferred_element_type=F32)
    ts_ref[...] = jnp.dot(ns, a_ref[...], preferred_element_type=F32) \
        + anorm * dm_ref[0:1, :] + bias_ref[...]
    td_ref[...] = jnp.dot(ns, b_ref[...], preferred_element_type=F32) \
        + anorm * dm_ref[1:2, :]
    tu_ref[:, 0:3] = u
    tu_ref[:, 3:_VW] = jnp.zeros((ns.shape[0], _VW - 3), F32)


_prep_call = pl.pallas_call(
    _prep_body,
    grid=(_NP // _BN,),
    in_specs=[
        pl.BlockSpec((_BN, _NS), lambda i: (i, 0)),
        pl.BlockSpec((_BN, 3), lambda i: (i, 0)),
        pl.BlockSpec((_NS, _NS), lambda i: (0, 0)),
        pl.BlockSpec((_NS, _NS), lambda i: (0, 0)),
        pl.BlockSpec((3, _NS), lambda i: (0, 0)),
        pl.BlockSpec((3, 3), lambda i: (0, 0)),
        pl.BlockSpec((3, 3), lambda i: (0, 0)),
        pl.BlockSpec((1, _NS), lambda i: (0, 0)),
    ],
    out_specs=[pl.BlockSpec((_BN, _NS), lambda i: (i, 0)),
               pl.BlockSpec((_BN, _NS), lambda i: (i, 0)),
               pl.BlockSpec((_BN, _VW), lambda i: (i, 0))],
    out_shape=[jax.ShapeDtypeStruct((_NP, _NS), F32),
               jax.ShapeDtypeStruct((_NP, _NS), F32),
               jax.ShapeDtypeStruct((_NP, _VW), F32)],
)


@functools.cache
def _gather1_call(h):
    def body_fn(ts_hbm, td_hbm, si3_hbm, di3_hbm, gs_hbm, gd_hbm,
                sidx, didx, bs, bd, sem):
        c = lax.axis_index("c")
        s = lax.axis_index("s")
        wid = s * _NCORES + c

        def body(g, carry):
            base = wid * _EPW + g * _GROUP
            gid = h * (_EH // _GROUP) + wid * _NGRP + g
            pltpu.sync_copy(si3_hbm.at[gid], sidx)
            pltpu.sync_copy(di3_hbm.at[gid], didx)
            cps = []
            for j in range(_NSUB):
                cps.append(pltpu.async_copy(
                    ts_hbm.at[sidx.at[j]],
                    bs.at[pl.ds(j * _SUB, _SUB)], sem))
                cps.append(pltpu.async_copy(
                    td_hbm.at[didx.at[j]],
                    bd.at[pl.ds(j * _SUB, _SUB)], sem))
            for cp in cps:
                cp.wait()
            pltpu.sync_copy(bs, gs_hbm.at[pl.ds(base, _GROUP)])
            pltpu.sync_copy(bd, gd_hbm.at[pl.ds(base, _GROUP)])
            return carry

        lax.fori_loop(0, _NGRP, body, 0)

    return pl.kernel(
        body_fn,
        out_type=(jax.ShapeDtypeStruct((_EH, _NS), F32),
                  jax.ShapeDtypeStruct((_EH, _NS), F32)),
        mesh=plsc.VectorSubcoreMesh(**_MESH),
        scratch_types=[
            pltpu.VMEM((_NSUB, _SUB), jnp.int32),
            pltpu.VMEM((_NSUB, _SUB), jnp.int32),
            pltpu.VMEM((_GROUP, _NS), F32),
            pltpu.VMEM((_GROUP, _NS), F32),
            pltpu.SemaphoreType.DMA,
        ],
    )


@functools.cache
def _gather2_call(h):
    def body_fn(tu_hbm, si3_hbm, us_hbm, sidx, bu, sem):
        c = lax.axis_index("c")
        s = lax.axis_index("s")
        wid = s * _NCORES + c

        def body(g, carry):
            base = wid * _EPW + g * _GROUP
            gid = h * (_EH // _GROUP) + wid * _NGRP + g
            pltpu.sync_copy(si3_hbm.at[gid], sidx)
            cps = []
            for j in range(_NSUB):
                cps.append(pltpu.async_copy(
                    tu_hbm.at[sidx.at[j]],
                    bu.at[pl.ds(j * _SUB, _SUB)], sem))
            for cp in cps:
                cp.wait()
            pltpu.sync_copy(bu, us_hbm.at[pl.ds(base, _GROUP)])
            return carry

        lax.fori_loop(0, _NGRP, body, 0)

    return pl.kernel(
        body_fn,
        out_type=jax.ShapeDtypeStruct((_EH, _VW), F32),
        mesh=plsc.VectorSubcoreMesh(**_MESH),
        scratch_types=[
            pltpu.VMEM((_NSUB, _SUB), jnp.int32),
            pltpu.VMEM((_GROUP, _VW), F32),
            pltpu.SemaphoreType.DMA,
        ],
        compiler_params=pltpu.CompilerParams(use_tc_tiling_on_sc=False),
    )


def _edge_body(gs_ref, gd_ref, us_ref, es_ref, ev_ref, c16_ref, dm2_ref,
               wg8_ref, wgb8_ref, wht_ref, wcm_ref, e1_ref, c3_ref,
               ms_ref, mv_ref):
    ev = ev_ref[...]
    vh = jnp.dot(ev, wht_ref[...], preferred_element_type=F32)
    cnorm = jnp.sqrt(jnp.sum(vh * vh, axis=-1, keepdims=True))
    q = jnp.dot(es_ref[...], c16_ref[...], preferred_element_type=F32)
    slin = gs_ref[...] + gd_ref[...] + q + cnorm * dm2_ref[...]
    so = jnp.maximum(slin, 0.0)
    gate = jax.nn.sigmoid(
        jnp.dot(so, wg8_ref[...], preferred_element_type=F32) + wgb8_ref[...])
    mult = (us_ref[...]
            + jnp.dot(ev, wcm_ref[...], preferred_element_type=F32)
            + c3_ref[...])
    gate16 = jnp.dot(gate, e1_ref[...], preferred_element_type=F32)
    ms_ref[...] = so
    mv_ref[...] = gate16 * mult


_edge_call = pl.pallas_call(
    _edge_body,
    grid=(_EH // _BE,),
    in_specs=[
        pl.BlockSpec((_BE, _NS), lambda i: (i, 0)),
        pl.BlockSpec((_BE, _NS), lambda i: (i, 0)),
        pl.BlockSpec((_BE, _VW), lambda i: (i, 0)),
        pl.BlockSpec((_BE, 16), lambda i: (i, 0)),
        pl.BlockSpec((_BE, 3), lambda i: (i, 0)),
        pl.BlockSpec((16, _NS), lambda i: (0, 0)),
        pl.BlockSpec((1, _NS), lambda i: (0, 0)),
        pl.BlockSpec((_NS, 8), lambda i: (0, 0)),
        pl.BlockSpec((1, 8), lambda i: (0, 0)),
        pl.BlockSpec((3, 3), lambda i: (0, 0)),
        pl.BlockSpec((3, _VW), lambda i: (0, 0)),
        pl.BlockSpec((8, _VW), lambda i: (0, 0)),
        pl.BlockSpec((1, _VW), lambda i: (0, 0)),
    ],
    out_specs=[pl.BlockSpec((_BE, _NS), lambda i: (i, 0)),
               pl.BlockSpec((_BE, _VW), lambda i: (i, 0))],
    out_shape=[jax.ShapeDtypeStruct((_EH, _NS), F32),
               jax.ShapeDtypeStruct((_EH, _VW), F32)],
)


@functools.cache
def _scatter1_call(h):
    def body_fn(m_hbm, di3_hbm, z_hbm, out_hbm, didx, buf, acc, sem):
        c = lax.axis_index("c")
        s = lax.axis_index("s")
        pltpu.sync_copy(z_hbm.at[pl.ds(s * _ROWS_PT, _ROWS_PT)],
                        acc.at[pl.ds(s * _ROWS_PT, _ROWS_PT)])
        plsc.subcore_barrier()
        base0 = c * (_EH // _NCORES) + s * _EPW

        def body(g, carry):
            base = base0 + g * _S1GROUP
            gid = (h * _EH + base0) // _S1GROUP + g
            pltpu.sync_copy(di3_hbm.at[gid], didx)
            pltpu.sync_copy(m_hbm.at[pl.ds(base, _S1GROUP)], buf)
            cps = []
            for j in range(_S1NSUB):
                cps.append(pltpu.async_copy(
                    buf.at[pl.ds(j * _S1SUB, _S1SUB)], acc.at[didx.at[j]],
                    sem, add=True))
            for cp in cps:
                cp.wait()
            return carry

        lax.fori_loop(0, _EPW // _S1GROUP, body, 0)
        plsc.subcore_barrier()
        pltpu.sync_copy(acc.at[pl.ds(s * _ROWS_PT, _ROWS_PT)],
                        out_hbm.at[c, pl.ds(s * _ROWS_PT, _ROWS_PT)])

    return pl.kernel(
        body_fn,
        out_type=jax.ShapeDtypeStruct((_NCORES, _NP, _NS), F32),
        mesh=plsc.VectorSubcoreMesh(**_MESH),
        scratch_types=[
            pltpu.VMEM((_S1NSUB, _S1SUB), jnp.int32),
            pltpu.VMEM((_S1GROUP, _NS), F32),
            pltpu.VMEM_SHARED((_NP, _NS), F32),
            pltpu.SemaphoreType.DMA,
        ],
    )


@functools.cache
def _scatter2_call(h):
    def body_fn(m_hbm, di3_hbm, z_hbm, out_hbm, didx, buf, acc, sem):
        c = lax.axis_index("c")
        s = lax.axis_index("s")
        pltpu.sync_copy(z_hbm.at[pl.ds(s * _ROWS_PT, _ROWS_PT)],
                        acc.at[pl.ds(s * _ROWS_PT, _ROWS_PT)])
        plsc.subcore_barrier()
        base0 = c * (_EH // _NCORES) + s * _EPW

        def body(g, carry):
            base = base0 + g * _GROUP
            gid = (h * _EH + base0) // _GROUP + g
            pltpu.sync_copy(di3_hbm.at[gid], didx)
            pltpu.sync_copy(m_hbm.at[pl.ds(base, _GROUP)], buf)
            cps = []
            for j in range(_NSUB):
                cps.append(pltpu.async_copy(
                    buf.at[pl.ds(j * _SUB, _SUB)], acc.at[didx.at[j]], sem,
                    add=True))
            for cp in cps:
                cp.wait()
            return carry

        lax.fori_loop(0, _NGRP, body, 0)
        plsc.subcore_barrier()
        pltpu.sync_copy(acc.at[pl.ds(s * _ROWS_PT, _ROWS_PT)],
                        out_hbm.at[c, pl.ds(s * _ROWS_PT, _ROWS_PT)])

    return pl.kernel(
        body_fn,
        out_type=jax.ShapeDtypeStruct((_NCORES, _NP, _VW), F32),
        mesh=plsc.VectorSubcoreMesh(**_MESH),
        scratch_types=[
            pltpu.VMEM((_NSUB, _SUB), jnp.int32),
            pltpu.VMEM((_GROUP, _VW), F32),
            pltpu.VMEM_SHARED((_NP, _VW), F32),
            pltpu.SemaphoreType.DMA,
        ],
        compiler_params=pltpu.CompilerParams(use_tc_tiling_on_sc=False),
    )


def _node_gvp(s, v9, a_ref, d_ref, b_ref, wg8_ref, wgb8_ref, bdh_ref,
              bdc_ref, sel_ref, selt_ref):
    vh9 = jnp.dot(v9, bdh_ref[...], preferred_element_type=F32)
    vn = jnp.sqrt(jnp.dot(vh9 * vh9, sel_ref[...],
                          preferred_element_type=F32))
    slin = (jnp.dot(s, a_ref[...], preferred_element_type=F32)
            + jnp.dot(vn, d_ref[...], preferred_element_type=F32)
            + b_ref[...])
    so = jnp.maximum(slin, 0.0)
    gate = jax.nn.sigmoid(
        jnp.dot(so, wg8_ref[...], preferred_element_type=F32)
        + wgb8_ref[...])[:, 0:3]
    gate9 = jnp.dot(gate, selt_ref[...], preferred_element_type=F32)
    vout = jnp.dot(v9, bdc_ref[...], preferred_element_type=F32) * gate9
    return so, vout


def _layernorm(x, w, b):
    mu = jnp.mean(x, axis=-1, keepdims=True)
    var = jnp.mean((x - mu) ** 2, axis=-1, keepdims=True)
    return (x - mu) / jnp.sqrt(var + 1e-5) * w + b


def _node_body(pa0_ref, pa1_ref,
               qa0_ref, qa1_ref, ns_ref, nv_ref,
               wht_ref, wvt_ref,
               ln1w_ref, ln1b_ref, ln2w_ref, ln2b_ref,
               a0_ref, d0_ref, b0_ref, wg0_ref, wgb0_ref, bdh0_ref, bdc0_ref,
               a1_ref, d1_ref, b1_ref, wg1_ref, wgb1_ref, bdh1_ref, bdc1_ref,
               sel_ref, selt_ref, os_ref, ov_ref):
    agg_s = pa0_ref[...] + pa1_ref[...] + ns_ref[...]
    pv = qa0_ref[...] + qa1_ref[...]
    nh = jnp.dot(nv_ref[...], wht_ref[...], preferred_element_type=F32)
    u = jnp.dot(nh, wvt_ref[...], preferred_element_type=F32)
    v9 = jnp.concatenate(
        [pv[:, 0:3], u * pv[:, 3:4], pv[:, 4:7]], axis=1)
    s1 = _layernorm(agg_s, ln1w_ref[...], ln1b_ref[...])
    rms = jnp.sqrt(jnp.mean(v9 * v9, axis=-1, keepdims=True) + 1e-8)
    v1 = v9 / rms
    s2, v2 = _node_gvp(s1, v1, a0_ref, d0_ref, b0_ref, wg0_ref, wgb0_ref,
                       bdh0_ref, bdc0_ref, sel_ref, selt_ref)
    s3, v3 = _node_gvp(s2, v2, a1_ref, d1_ref, b1_ref, wg1_ref, wgb1_ref,
                       bdh1_ref, bdc1_ref, sel_ref, selt_ref)
    o_s = s1 + s3
    o_v = v1 + v3
    os_ref[...] = _layernorm(o_s, ln2w_ref[...], ln2b_ref[...])
    rms2 = jnp.sqrt(jnp.mean(o_v * o_v, axis=-1, keepdims=True) + 1e-8)
    ov_ref[...] = o_v / rms2


def _full(shape):
    return pl.BlockSpec(shape, lambda i: tuple(0 for _ in shape))


_node_call = pl.pallas_call(
    _node_body,
    grid=(_NP // _BN,),
    in_specs=[
        pl.BlockSpec((_BN, _NS), lambda i: (i, 0)),
        pl.BlockSpec((_BN, _NS), lambda i: (i, 0)),
        pl.BlockSpec((_BN, _VW), lambda i: (i, 0)),
        pl.BlockSpec((_BN, _VW), lambda i: (i, 0)),
        pl.BlockSpec((_BN, _NS), lambda i: (i, 0)),
        pl.BlockSpec((_BN, 3), lambda i: (i, 0)),
        _full((3, 3)), _full((3, 3)),
        _full((1, _NS)), _full((1, _NS)), _full((1, _NS)), _full((1, _NS)),
        _full((_NS, _NS)), _full((3, _NS)), _full((1, _NS)),
        _full((_NS, 8)), _full((1, 8)), _full((9, 9)), _full((9, 9)),
        _full((_NS, _NS)), _full((3, _NS)), _full((1, _NS)),
        _full((_NS, 8)), _full((1, 8)), _full((9, 9)), _full((9, 9)),
        _full((9, 3)), _full((3, 9)),
    ],
    out_specs=[pl.BlockSpec((_BN, _NS), lambda i: (i, 0)),
               pl.BlockSpec((_BN, 9), lambda i: (i, 0))],
    out_shape=[jax.ShapeDtypeStruct((_NP, _NS), F32),
               jax.ShapeDtypeStruct((_NP, 9), F32)],
)

_SEL = np.zeros((9, 3), np.float32)
for _i in range(3):
    for _k in range(3):
        _SEL[3 * _i + _k, _i] = 1.0


def _blockdiag3(w):
    z = jnp.zeros((9, 9), F32)
    for i in range(3):
        z = z.at[3 * i:3 * i + 3, 3 * i:3 * i + 3].set(w)
    return z


def _pad8(w3):
    return jnp.zeros((w3.shape[1], 8), F32).at[:, 0:3].set(w3.T)


def kernel(node_s, node_v, edge_s, edge_v, msg_Wh, msg_WV, msg_Ws_w,
           msg_Ws_b, msg_Wg_w, msg_Wg_b, ff0_Wh, ff0_WV, ff0_Ws_w, ff0_Ws_b,
           ff0_Wg_w, ff0_Wg_b, ff1_Wh, ff1_WV, ff1_Ws_w, ff1_Ws_b, ff1_Wg_w,
           ff1_Wg_b, ln1_w, ln1_b, ln2_w, ln2_b, edge_index):
    ns_p = jnp.zeros((_NP, _NS), F32).at[:_N].set(node_s)
    nv_p = jnp.zeros((_NP, 3), F32).at[:_N].set(node_v.reshape(_N, 3))
    wst = msg_Ws_w.T
    a_w, b_w, c16, dm = wst[0:128], wst[128:256], wst[256:272], wst[272:275]
    wht = msg_Wh.T
    wc = msg_Wh.T @ msg_WV.T
    ts, td, tu = _prep_call(ns_p, nv_p, a_w, b_w, dm, wht, msg_WV.T,
                            msg_Ws_b[None])
    src3 = edge_index[0].reshape(_NGTOT, _NSUB, _SUB)
    dst3 = edge_index[1].reshape(_NGTOT, _NSUB, _SUB)
    ev3 = edge_v.reshape(_E, 3)
    wgb8 = jnp.zeros((1, 8), F32).at[0, 0:3].set(msg_Wg_b)
    wg8 = _pad8(msg_Wg_w)
    zeros_s = jnp.zeros((_NP, _NS), F32)
    zeros_v = jnp.zeros((_NP, _VW), F32)
    wcm = jnp.zeros((3, _VW), F32).at[:, 4:7].set(wc)
    c3 = jnp.zeros((1, _VW), F32).at[0, 3].set(1.0)
    e1 = jnp.zeros((8, _VW), F32)
    e1 = e1.at[0, 0:3].set(1.0).at[1, 3].set(1.0).at[2, 4:7].set(1.0)

    gath, usv, msv, mvv, ps, pv = [], [], [], [], [], []
    for hh in range(_NH):
        gath.append(_gather1_call(hh)(ts, td, src3, dst3))
        usv.append(_gather2_call(hh)(tu, src3))
    for hh in range(_NH):
        gs, gd = gath[hh]
        sl = slice(hh * _EH, (hh + 1) * _EH)
        m_s, m_v = _edge_call(gs, gd, usv[hh], edge_s[sl], ev3[sl], c16,
                              dm[2:3], wg8, wgb8, wht, wcm, e1, c3)
        msv.append(m_s)
        mvv.append(m_v)
    dst3s = edge_index[1].reshape(_E // _S1GROUP, _S1NSUB, _S1SUB)
    for hh in range(_NH):
        ps.append(_scatter1_call(hh)(msv[hh], dst3s, zeros_s))
        pv.append(_scatter2_call(hh)(mvv[hh], dst3, zeros_v))

    def ffw(ws_w, ws_b, wg_w, wg_b, wh, wv):
        t = ws_w.T
        wgb = jnp.zeros((1, 8), F32).at[0, 0:3].set(wg_b)
        return (t[0:128], t[128:131], ws_b[None], _pad8(wg_w), wgb,
                _blockdiag3(wh.T), _blockdiag3(wh.T @ wv.T))

    sel = jnp.asarray(_SEL)
    out_s, out_v9 = _node_call(
        ps[0][0], ps[0][1],
        pv[0][0], pv[0][1], ns_p, nv_p,
        wht, msg_WV.T,
        ln1_w[None], ln1_b[None], ln2_w[None], ln2_b[None],
        *ffw(ff0_Ws_w, ff0_Ws_b, ff0_Wg_w, ff0_Wg_b, ff0_Wh, ff0_WV),
        *ffw(ff1_Ws_w, ff1_Ws_b, ff1_Wg_w, ff1_Wg_b, ff1_Wh, ff1_WV),
        sel, sel.T)
    return out_s[:_N], out_v9[:_N].reshape(_N, 3, 3)
